# baseline (speedup 1.0000x reference)
import functools

import jax
import jax.numpy as jnp
from jax import lax
from jax.experimental import pallas as pl
from jax.experimental.pallas import tpu as pltpu

EMB = 64
AUG = 80
SROWS = 32
F32 = jnp.float32


def _relu(x):
    return jnp.maximum(x, 0.0)


def _ln_rows(x, g, b):
    m = jnp.mean(x, axis=-1, keepdims=True)
    v = jnp.mean(x * x, axis=-1, keepdims=True) - m * m
    return (x - m) * lax.rsqrt(v + 1e-5) * g + b


def _stats_kernel(x_ref, o_ref):
    x = x_ref[...]
    s = jnp.sum(x, axis=0, keepdims=True)
    q = jnp.sum(x * x, axis=0, keepdims=True)
    o_ref[...] = jnp.concatenate([s, q], axis=0)


def _col_stats(x):
    return pl.pallas_call(
        _stats_kernel,
        out_shape=jax.ShapeDtypeStruct((2, x.shape[1]), F32),
    )(x)


def _prologue_kernel(x_ref, m_ref, s_ref, bb_ref, w1_ref, b1_ref, w2_ref,
                     b2_ref, p1w_ref, p1b_ref, p2w_ref, p2b_ref,
                     emb_ref, p1_ref, p2_ref):
    x = x_ref[...]
    xn = (x - m_ref[...]) * s_ref[...] + bb_ref[...]
    h = _relu(jnp.dot(xn, w1_ref[...], preferred_element_type=F32) + b1_ref[...])
    h = _relu(jnp.dot(h, w2_ref[...], preferred_element_type=F32) + b2_ref[...])
    emb_ref[...] = h
    p1_ref[...] = jnp.dot(h, p1w_ref[...], preferred_element_type=F32) + p1b_ref[...]
    p2_ref[...] = jnp.dot(h, p2w_ref[...], preferred_element_type=F32) + p2b_ref[...]


def _prologue(x, bn_g, bn_b, W1, b1, W2, b2, P1w, P1b, P2w, P2b, chunk=2000):
    N, D = x.shape
    st = _col_stats(x)
    m = st[0] / N
    var = st[1] / N - m * m
    scale = lax.rsqrt(var + 1e-5) * bn_g
    grid = (N // chunk,)
    bs_x = pl.BlockSpec((chunk, D), lambda i: (i, 0))
    bs_row = pl.BlockSpec((1, D), lambda i: (0, 0))
    bs_w1 = pl.BlockSpec((D, EMB), lambda i: (0, 0))
    bs_e = pl.BlockSpec((1, EMB), lambda i: (0, 0))
    bs_w = pl.BlockSpec((EMB, EMB), lambda i: (0, 0))
    bs_o = pl.BlockSpec((chunk, EMB), lambda i: (i, 0))
    out_sh = jax.ShapeDtypeStruct((N, EMB), F32)
    return pl.pallas_call(
        _prologue_kernel,
        grid=grid,
        in_specs=[bs_x, bs_row, bs_row, bs_row, bs_w1, bs_e, bs_w, bs_e,
                  bs_w, bs_e, bs_w, bs_e],
        out_specs=[bs_o, bs_o, bs_o],
        out_shape=[out_sh, out_sh, out_sh],
    )(x, m[None, :], scale[None, :], bn_b[None, :], W1, b1[None, :], W2,
      b2[None, :], P1w, P1b[None, :], P2w, P2b[None, :])


def _post_kernel(hs_ref, right_ref, wa_ref, pg_ref, pb_ref,
                 o1a_ref, o1b_ref, o1bias_ref, o2w_ref, o2b_ref,
                 pw_ref, pbias_ref, new_ref, proj_ref, sum_ref):
    agg = jnp.dot(hs_ref[...], wa_ref[...], preferred_element_type=F32)
    ln = _ln_rows(agg, pg_ref[...], pb_ref[...])
    t = (jnp.dot(ln, o1a_ref[...], preferred_element_type=F32)
         + jnp.dot(right_ref[...], o1b_ref[...], preferred_element_type=F32)
         + o1bias_ref[...])
    t = _relu(t)
    new = jnp.dot(t, o2w_ref[...], preferred_element_type=F32) + o2b_ref[...]
    new_ref[...] = new
    proj_ref[...] = jnp.dot(new, pw_ref[...], preferred_element_type=F32) + pbias_ref[...]

    @pl.when(pl.program_id(0) == 0)
    def _init():
        sum_ref[...] = jnp.zeros_like(sum_ref)
    sum_ref[...] += jnp.sum(new, axis=0, keepdims=True)


def _post(hs_aug, right, p, Pw, Pb, chunk=2000):
    N = right.shape[0]
    W_aug = jnp.concatenate(
        [p['fin_W'], p['fin_b'][None, :], jnp.zeros((AUG - EMB - 1, EMB), F32)], axis=0)
    o1a = p['o1_W'][:EMB]
    o1b = p['o1_W'][EMB:]
    grid = (N // chunk,)
    bs_hs = pl.BlockSpec((chunk, AUG), lambda i: (i, 0))
    bs_r = pl.BlockSpec((chunk, EMB), lambda i: (i, 0))
    bs_wa = pl.BlockSpec((AUG, EMB), lambda i: (0, 0))
    bs_e = pl.BlockSpec((1, EMB), lambda i: (0, 0))
    bs_w = pl.BlockSpec((EMB, EMB), lambda i: (0, 0))
    bs_o = pl.BlockSpec((chunk, EMB), lambda i: (i, 0))
    bs_sum = pl.BlockSpec((1, EMB), lambda i: (0, 0))
    return pl.pallas_call(
        _post_kernel,
        grid=grid,
        in_specs=[bs_hs, bs_r, bs_wa, bs_e, bs_e, bs_w, bs_w, bs_e, bs_w,
                  bs_e, bs_w, bs_e],
        out_specs=[bs_o, bs_o, bs_sum],
        out_shape=[jax.ShapeDtypeStruct((N, EMB), F32),
                   jax.ShapeDtypeStruct((N, EMB), F32),
                   jax.ShapeDtypeStruct((1, EMB), F32)],
    )(hs_aug, right, W_aug, p['post_g'][None, :], p['post_b'][None, :],
      o1a, o1b, p['o1_b'][None, :], p['o2_W'], p['o2_b'][None, :],
      Pw, Pb[None, :])


def _small_edge_kernel(a_ref, b_ref, src_ref, dst_ref, ev_ref, w0_ref,
                       g_ref, bln_ref, hs_ref):
    C = src_ref.shape[0]
    io = lax.broadcasted_iota(jnp.int32, (C, SROWS), 1)
    oh_src = (src_ref[...] == io).astype(F32)
    oh_dst = (dst_ref[...] == io).astype(F32)
    h = (jnp.dot(oh_dst, a_ref[...], preferred_element_type=F32)
         + jnp.dot(oh_src, b_ref[...], preferred_element_type=F32)
         + ev_ref[...] * w0_ref[...])
    h = _relu(_ln_rows(h, g_ref[...], bln_ref[...]))
    aug = jnp.concatenate(
        [h, jnp.ones((C, 1), F32), jnp.zeros((C, AUG - EMB - 1), F32)], axis=-1)
    acc = lax.dot_general(oh_dst, aug, (((0,), (0,)), ((), ())),
                          preferred_element_type=F32)

    @pl.when(pl.program_id(0) == 0)
    def _init():
        hs_ref[...] = jnp.zeros_like(hs_ref)
    hs_ref[...] += acc


def _small_edge(A32, B32, src, dst, ev, w0p, ln_g, ln_b, chunk=3400):
    E = src.shape[0]
    grid = (E // chunk,)
    bs_t = pl.BlockSpec((SROWS, EMB), lambda i: (0, 0))
    bs_i = pl.BlockSpec((chunk, 1), lambda i: (i, 0))
    bs_row = pl.BlockSpec((1, EMB), lambda i: (0, 0))
    bs_hs = pl.BlockSpec((SROWS, AUG), lambda i: (0, 0))
    return pl.pallas_call(
        _small_edge_kernel,
        grid=grid,
        in_specs=[bs_t, bs_t, bs_i, bs_i, bs_i, bs_row, bs_row, bs_row],
        out_specs=bs_hs,
        out_shape=jax.ShapeDtypeStruct((SROWS, AUG), F32),
    )(A32, B32, src, dst, ev, w0p, ln_g[None, :], ln_b[None, :])


def _small_post_kernel(hs_ref, right_ref, wa_ref, pg_ref, pb_ref,
                       o1a_ref, o1b_ref, o1bias_ref, o2w_ref, o2b_ref,
                       aw_ref, ab_ref, bw_ref, new_ref, a32_ref, b32_ref):
    agg = jnp.dot(hs_ref[...], wa_ref[...], preferred_element_type=F32)
    ln = _ln_rows(agg, pg_ref[...], pb_ref[...])
    t = _relu(jnp.dot(ln, o1a_ref[...], preferred_element_type=F32)
              + jnp.dot(right_ref[...], o1b_ref[...], preferred_element_type=F32)
              + o1bias_ref[...])
    new = jnp.dot(t, o2w_ref[...], preferred_element_type=F32) + o2b_ref[...]
    new_ref[...] = new
    a32_ref[...] = jnp.dot(new, aw_ref[...], preferred_element_type=F32) + ab_ref[...]
    b32_ref[...] = jnp.dot(new, bw_ref[...], preferred_element_type=F32)


def _small_post(hs32, right32, p, Aw, Ab, Bw):
    W_aug = jnp.concatenate(
        [p['fin_W'], p['fin_b'][None, :], jnp.zeros((AUG - EMB - 1, EMB), F32)], axis=0)
    sh = jax.ShapeDtypeStruct((SROWS, EMB), F32)
    return pl.pallas_call(
        _small_post_kernel,
        out_shape=[sh, sh, sh],
    )(hs32, right32, W_aug, p['post_g'][None, :], p['post_b'][None, :],
      p['o1_W'][:EMB], p['o1_W'][EMB:], p['o1_b'][None, :], p['o2_W'],
      p['o2_b'][None, :], Aw, Ab[None, :], Bw)


def _tconv_kernel(x_ref, xs_ref, src_ref, dst_ref, ev_ref,
                  qw_ref, qb_ref, kw_ref, kb_ref, vw_ref, vb_ref, te_ref,
                  skw_ref, skb_ref, sowx_ref, sowa_ref, sows_ref, sob_ref,
                  out_ref):
    x = x_ref[...]
    C = src_ref.shape[0]
    io = lax.broadcasted_iota(jnp.int32, (C, SROWS), 1)
    oh_src = (src_ref[...] == io).astype(F32)
    oh_dst = (dst_ref[...] == io).astype(F32)
    q = jnp.dot(x, qw_ref[...], preferred_element_type=F32) + qb_ref[...]
    k = jnp.dot(x, kw_ref[...], preferred_element_type=F32) + kb_ref[...]
    v = jnp.dot(x, vw_ref[...], preferred_element_type=F32) + vb_ref[...]
    e = ev_ref[...] * te_ref[...]
    kj = jnp.dot(oh_src, k, preferred_element_type=F32) + e
    qd = jnp.dot(oh_dst, q, preferred_element_type=F32)
    vj = jnp.dot(oh_src, v, preferred_element_type=F32) + e
    att = jnp.zeros((SROWS, EMB), F32)
    pad = ev_ref[...] * 0.0
    for h in range(4):
        sl = slice(h * 16, (h + 1) * 16)
        alpha = jnp.sum(qd[:, sl] * kj[:, sl], axis=-1, keepdims=True) * 0.25
        big = jnp.where(oh_dst > 0.0, alpha + pad, -1e30)
        amax = jnp.max(big, axis=0, keepdims=True)
        asub = jnp.sum(oh_dst * amax, axis=1, keepdims=True)
        ex = jnp.exp(alpha - asub)
        den = lax.dot_general(oh_dst, ex, (((0,), (0,)), ((), ())),
                              preferred_element_type=F32)
        dend = jnp.sum(oh_dst * den.T, axis=1, keepdims=True) + 1e-16
        a = ex / dend
        outh = vj[:, sl] * a
        aggh = lax.dot_general(oh_dst, outh, (((0,), (0,)), ((), ())),
                               preferred_element_type=F32)
        att = att + jnp.pad(aggh, ((0, 0), (h * 16, EMB - (h + 1) * 16)))
    att = att + jnp.dot(x, skw_ref[...], preferred_element_type=F32) + skb_ref[...]
    satt = _relu(jnp.dot(x, sowx_ref[...], preferred_element_type=F32)
                 + jnp.dot(att, sowa_ref[...], preferred_element_type=F32)
                 + xs_ref[...] * sows_ref[...] + sob_ref[...])
    mask = (lax.broadcasted_iota(jnp.int32, (SROWS, 1), 0) < 17).astype(F32)
    out_ref[...] = jnp.sum(satt * mask, axis=0, keepdims=True) * (1.0 / 17.0)


def _tconv_satt_mean(sep32, xs32, src, dst, ev, p):
    return pl.pallas_call(
        _tconv_kernel,
        out_shape=jax.ShapeDtypeStruct((1, EMB), F32),
    )(sep32, xs32, src, dst, ev,
      p['tq_W'], p['tq_b'][None, :], p['tk_W'], p['tk_b'][None, :],
      p['tv_W'], p['tv_b'][None, :], p['te_W'][0][None, :],
      p['tskip_W'], p['tskip_b'][None, :],
      p['so_W'][:EMB], p['so_W'][EMB:2 * EMB], p['so_W'][2 * EMB][None, :],
      p['so_b'][None, :])


def _final_kernel(sa_ref, ra_ref, ca_ref, w1a_ref, w1b_ref, w1c_ref,
                  b1_ref, w2_ref, b2_ref, out_ref):
    h = _relu(jnp.dot(sa_ref[...], w1a_ref[...], preferred_element_type=F32)
              + jnp.dot(ra_ref[...], w1b_ref[...], preferred_element_type=F32)
              + jnp.dot(ca_ref[...], w1c_ref[...], preferred_element_type=F32)
              + b1_ref[...])
    z = jnp.dot(h, w2_ref[...], preferred_element_type=F32) + b2_ref[...]
    out_ref[...] = 1.0 / (1.0 + jnp.exp(-z))


def _final(sa, ra, ca, p):
    return pl.pallas_call(
        _final_kernel,
        out_shape=jax.ShapeDtypeStruct((1, 1), F32),
    )(sa, ra, ca, p['out_W1'][:EMB], p['out_W1'][EMB:2 * EMB],
      p['out_W1'][2 * EMB:], p['out_b1'][None, :], p['out_W2'],
      p['out_b2'][None, :])


def _proj_kernel(x_ref, w_ref, b_ref, o_ref):
    o_ref[...] = (jnp.dot(x_ref[...], w_ref[...], preferred_element_type=F32)
                  + b_ref[...])


def _proj32(x32, W, bvec):
    return pl.pallas_call(
        _proj_kernel, out_shape=jax.ShapeDtypeStruct((SROWS, EMB), F32),
    )(x32, W, bvec[None, :])


def _rattsum_kernel(x_ref, w_ref, b_ref, o_ref):
    @pl.when(pl.program_id(0) == 0)
    def _init():
        o_ref[...] = jnp.zeros_like(o_ref)
    o_ref[...] += jnp.sum(
        _relu(jnp.dot(x_ref[...], w_ref[...], preferred_element_type=F32)
              + b_ref[...]), axis=0, keepdims=True)


def _relu_matsum(x, W, bvec, chunk=2000):
    N = x.shape[0]
    return pl.pallas_call(
        _rattsum_kernel,
        grid=(N // chunk,),
        in_specs=[pl.BlockSpec((chunk, EMB), lambda i: (i, 0)),
                  pl.BlockSpec((EMB, EMB), lambda i: (0, 0)),
                  pl.BlockSpec((1, EMB), lambda i: (0, 0))],
        out_specs=pl.BlockSpec((1, EMB), lambda i: (0, 0)),
        out_shape=jax.ShapeDtypeStruct((1, EMB), F32),
    )(x, W, bvec[None, :])


def _edge_fold(ev_2xE_pad, count, g, b, e_W):
    st = [_col_stats(ev_2xE_pad[d]) for d in range(ev_2xE_pad.shape[0])]
    s_list, c_parts = [], []
    for d, std in enumerate(st):
        tot = jnp.sum(std[0])
        totq = jnp.sum(std[1])
        m = tot / count
        v = totq / count - m * m
        s = g[d] * lax.rsqrt(v + 1e-5)
        s_list.append(s)
        c_parts.append((b[d] - m * s) * e_W[d])
    Wp = jnp.stack([e_W[d] * s_list[d] for d in range(len(st))])
    c = sum(c_parts)
    return Wp, c


def _big_edge_jnp(A, B, src, dst, ev0, ev1, w0p, w1p, g, b):
    h = A[dst] + B[src] + ev0[:, None] * w0p + ev1[:, None] * w1p
    h = _relu(_ln_rows(h, g, b))
    aug = jnp.concatenate([h, jnp.ones((h.shape[0], 1), F32)], axis=-1)
    hs = jnp.zeros((A.shape[0], EMB + 1), F32).at[dst].add(aug)
    return jnp.pad(hs, ((0, 0), (0, AUG - EMB - 1)))


def kernel(x_rows, x_cols, x_sepas, edge_index_rowcols, edge_vals_rowcols,
           edge_index_sepa_cols, edge_vals_sepa_cols, edge_index_sepa_rows,
           edge_vals_sepa_rows, edge_index_sepa_self, edge_vals_sepa_self,
           params):
    p = params
    ei_rc = edge_index_rowcols.astype(jnp.int32)
    ei_sc = edge_index_sepa_cols.astype(jnp.int32)
    ei_sr = edge_index_sepa_rows.astype(jnp.int32)
    ei_ss = edge_index_sepa_self.astype(jnp.int32)

    E_RC = ei_rc.shape[1]
    E_SC = ei_sc.shape[1]
    E_SR = ei_sr.shape[1]

    evT_rc = edge_vals_rowcols.T.reshape(2, E_RC // 128, 128)
    Wp_rc, c_rc = _edge_fold(evT_rc, E_RC, p['en_rowcols_g'],
                             p['en_rowcols_b'], p['c2r']['e_W'])
    Wp_rc2, c_rc2 = _edge_fold(evT_rc, E_RC, p['en_rowcols_g'],
                               p['en_rowcols_b'], p['r2c']['e_W'])

    def _pad128(v):
        E = v.shape[0]
        R = -(-E // 128) * 128
        return jnp.pad(v, (0, R - E)).reshape(1, R // 128, 128)

    Wp_sc, c_sc = _edge_fold(_pad128(edge_vals_sepa_cols[:, 0]), E_SC,
                             p['en_sepas_g'], p['en_sepas_b'], p['c2s']['e_W'])
    Wp_sr, c_sr = _edge_fold(_pad128(edge_vals_sepa_rows[:, 0]), E_SR,
                             p['en_rows_g'], p['en_rows_b'], p['s2r']['e_W'])
    Wp_r2s, c_r2s = _edge_fold(_pad128(edge_vals_sepa_rows[:, 0]), E_SR,
                               p['en_rows_g'], p['en_rows_b'], p['r2s']['e_W'])

    row0, A_c2r, _ = _prologue(
        x_rows, p['row_bn_g'], p['row_bn_b'], p['row_W1'], p['row_b1'],
        p['row_W2'], p['row_b2'],
        p['c2r']['l_W'], p['c2r']['l_b'] + c_rc,
        jnp.zeros((EMB, EMB), F32), jnp.zeros((EMB,), F32))
    col0, B_c2r, A_r2c = _prologue(
        x_cols, p['col_bn_g'], p['col_bn_b'], p['col_W1'], p['col_b1'],
        p['col_W2'], p['col_b2'],
        p['c2r']['r_W'], jnp.zeros((EMB,), F32),
        p['r2c']['l_W'], p['r2c']['l_b'] + c_rc2)

    ev0 = edge_vals_rowcols[:, 0]
    ev1 = edge_vals_rowcols[:, 1]

    hs_c2r = _big_edge_jnp(A_c2r, B_c2r, ei_rc[1], ei_rc[0], ev0, ev1,
                           Wp_rc[0], Wp_rc[1],
                           p['c2r']['fin_g'], p['c2r']['fin_bln'])
    row1, B_r2c, _ = _post(hs_c2r, row0, p['c2r'],
                           p['r2c']['r_W'], jnp.zeros((EMB,), F32))

    hs_r2c = _big_edge_jnp(A_r2c, B_r2c, ei_rc[0], ei_rc[1], ev0, ev1,
                           Wp_rc2[0], Wp_rc2[1],
                           p['r2c']['fin_g'], p['r2c']['fin_bln'])
    col1, _, colsum = _post(hs_r2c, col0, p['r2c'],
                            jnp.zeros((EMB, EMB), F32), jnp.zeros((EMB,), F32))

    sep_b = p['sepa_ln_b'][None, :]
    sep0_row = _relu(_relu(sep_b @ p['sepa_W1'] + p['sepa_b1'])
                     @ p['sepa_W2'] + p['sepa_b2'])
    sep0 = jnp.broadcast_to(sep0_row, (SROWS, EMB)) * (
        (jnp.arange(SROWS) < 17).astype(F32)[:, None])
    A32_c2s = _proj32(sep0, p['c2s']['l_W'], p['c2s']['l_b'] + c_sc)
    B32_c2s = _proj32(jnp.pad(col1[:17], ((0, SROWS - 17), (0, 0))),
                      p['c2s']['r_W'], jnp.zeros((EMB,), F32))

    src_sc = ei_sc[1][:, None]
    dst_sc = ei_sc[0][:, None]
    hs_c2s = _small_edge(A32_c2s, B32_c2s, src_sc, dst_sc,
                         edge_vals_sepa_cols, Wp_sc[0][None, :],
                         p['c2s']['fin_g'], p['c2s']['fin_bln'])
    sep1, _, B32_s2r = _small_post(
        hs_c2s, sep0, p['c2s'],
        jnp.zeros((EMB, EMB), F32), jnp.zeros((EMB,), F32), p['s2r']['r_W'])
    row1_17 = jnp.pad(row1[:17], ((0, SROWS - 17), (0, 0)))
    A32_s2r = _proj32(row1_17, p['s2r']['l_W'], p['s2r']['l_b'] + c_sr)

    src_sr = ei_sr[0][:, None]
    dst_sr = ei_sr[1][:, None]
    hs_s2r17 = _small_edge(A32_s2r, B32_s2r, src_sr, dst_sr,
                           edge_vals_sepa_rows, Wp_sr[0][None, :],
                           p['s2r']['fin_g'], p['s2r']['fin_bln'])
    hs_s2r = jnp.concatenate(
        [hs_s2r17[:17], jnp.zeros((row1.shape[0] - 17, AUG), F32)], axis=0)
    row2, _, _ = _post(hs_s2r, row1, p['s2r'],
                       jnp.zeros((EMB, EMB), F32), jnp.zeros((EMB,), F32))

    row2_17 = jnp.pad(row2[:17], ((0, SROWS - 17), (0, 0)))
    A32_r2s = _proj32(sep1, p['r2s']['l_W'], p['r2s']['l_b'] + c_r2s)
    B32_r2s = _proj32(row2_17, p['r2s']['r_W'], jnp.zeros((EMB,), F32))

    hs_r2s = _small_edge(A32_r2s, B32_r2s, dst_sr, src_sr,
                         edge_vals_sepa_rows, Wp_r2s[0][None, :],
                         p['r2s']['fin_g'], p['r2s']['fin_bln'])
    sep2, _, _ = _small_post(hs_r2s, sep1, p['r2s'],
                             jnp.zeros((EMB, EMB), F32),
                             jnp.zeros((EMB,), F32),
                             jnp.zeros((EMB, EMB), F32))

    E_SS = ei_ss.shape[1]
    PSS = -(-E_SS // 8) * 8
    src_ss = jnp.pad(ei_ss[0], (0, PSS - E_SS),
                     constant_values=SROWS - 1)[:, None]
    dst_ss = jnp.pad(ei_ss[1], (0, PSS - E_SS),
                     constant_values=SROWS - 1)[:, None]
    ev_ss = jnp.pad(edge_vals_sepa_self[:, 0], (0, PSS - E_SS))[:, None]
    xs32 = jnp.pad(x_sepas, ((0, SROWS - 17), (0, 0)))
    sattmean = _tconv_satt_mean(sep2, xs32, src_ss, dst_ss, ev_ss, p)

    N = row2.shape[0]
    rattsum = _relu_matsum(row2, p['ro_W'], p['ro_b'])

    return _final(sattmean, rattsum / N, colsum / N, p)


# speedup vs baseline: 1.4650x; 1.4650x over previous
import functools

import jax
import jax.numpy as jnp
from jax import lax
from jax.experimental import pallas as pl
from jax.experimental.pallas import tpu as pltpu
from jax.experimental.pallas import tpu_sc as plsc

EMB = 64
AUG = 128
SROWS = 32
F32 = jnp.float32


def _relu(x):
    return jnp.maximum(x, 0.0)


def _ln_rows(x, g, b):
    m = jnp.mean(x, axis=-1, keepdims=True)
    v = jnp.mean(x * x, axis=-1, keepdims=True) - m * m
    return (x - m) * lax.rsqrt(v + 1e-5) * g + b


def _stats_kernel(x_ref, o_ref):
    x = x_ref[...]
    s = jnp.sum(x, axis=0, keepdims=True)
    q = jnp.sum(x * x, axis=0, keepdims=True)
    o_ref[...] = jnp.concatenate([s, q], axis=0)


def _col_stats(x):
    return pl.pallas_call(
        _stats_kernel,
        out_shape=jax.ShapeDtypeStruct((2, x.shape[1]), F32),
    )(x)


def _prologue_kernel(x_ref, m_ref, s_ref, bb_ref, w1_ref, b1_ref, w2_ref,
                     b2_ref, p1w_ref, p1b_ref, p2w_ref, p2b_ref,
                     emb_ref, p1_ref, p2_ref):
    x = x_ref[...]
    xn = (x - m_ref[...]) * s_ref[...] + bb_ref[...]
    h = _relu(jnp.dot(xn, w1_ref[...], preferred_element_type=F32) + b1_ref[...])
    h = _relu(jnp.dot(h, w2_ref[...], preferred_element_type=F32) + b2_ref[...])
    emb_ref[...] = h
    p1_ref[...] = jnp.dot(h, p1w_ref[...], preferred_element_type=F32) + p1b_ref[...]
    p2_ref[...] = jnp.dot(h, p2w_ref[...], preferred_element_type=F32) + p2b_ref[...]


def _prologue(x, bn_g, bn_b, W1, b1, W2, b2, P1w, P1b, P2w, P2b, chunk=2000):
    N, D = x.shape
    st = _col_stats(x)
    m = st[0] / N
    var = st[1] / N - m * m
    scale = lax.rsqrt(var + 1e-5) * bn_g
    grid = (N // chunk,)
    bs_x = pl.BlockSpec((chunk, D), lambda i: (i, 0))
    bs_row = pl.BlockSpec((1, D), lambda i: (0, 0))
    bs_w1 = pl.BlockSpec((D, EMB), lambda i: (0, 0))
    bs_e = pl.BlockSpec((1, EMB), lambda i: (0, 0))
    bs_w = pl.BlockSpec((EMB, EMB), lambda i: (0, 0))
    bs_wp = pl.BlockSpec((EMB, AUG), lambda i: (0, 0))
    bs_ep = pl.BlockSpec((1, AUG), lambda i: (0, 0))
    bs_o = pl.BlockSpec((chunk, EMB), lambda i: (i, 0))
    bs_op = pl.BlockSpec((chunk, AUG), lambda i: (i, 0))
    out_sh = jax.ShapeDtypeStruct((N, EMB), F32)
    out_shp = jax.ShapeDtypeStruct((N, AUG), F32)
    padw = lambda W: jnp.pad(W, ((0, 0), (0, AUG - EMB)))
    padb = lambda b: jnp.pad(b, (0, AUG - EMB))
    return pl.pallas_call(
        _prologue_kernel,
        grid=grid,
        in_specs=[bs_x, bs_row, bs_row, bs_row, bs_w1, bs_e, bs_w, bs_e,
                  bs_wp, bs_ep, bs_wp, bs_ep],
        out_specs=[bs_o, bs_op, bs_op],
        out_shape=[out_sh, out_shp, out_shp],
    )(x, m[None, :], scale[None, :], bn_b[None, :], W1, b1[None, :], W2,
      b2[None, :], padw(P1w), padb(P1b)[None, :], padw(P2w),
      padb(P2b)[None, :])


def _post_kernel(hs_ref, right_ref, wa_ref, pg_ref, pb_ref,
                 o1a_ref, o1b_ref, o1bias_ref, o2w_ref, o2b_ref,
                 pw_ref, pbias_ref, new_ref, proj_ref, sum_ref):
    agg = jnp.dot(hs_ref[...], wa_ref[...], preferred_element_type=F32)
    ln = _ln_rows(agg, pg_ref[...], pb_ref[...])
    t = (jnp.dot(ln, o1a_ref[...], preferred_element_type=F32)
         + jnp.dot(right_ref[...], o1b_ref[...], preferred_element_type=F32)
         + o1bias_ref[...])
    t = _relu(t)
    new = jnp.dot(t, o2w_ref[...], preferred_element_type=F32) + o2b_ref[...]
    new_ref[...] = new
    proj_ref[...] = jnp.dot(new, pw_ref[...], preferred_element_type=F32) + pbias_ref[...]

    @pl.when(pl.program_id(0) == 0)
    def _init():
        sum_ref[...] = jnp.zeros_like(sum_ref)
    sum_ref[...] += jnp.sum(new, axis=0, keepdims=True)


def _post(hs_aug, right, p, Pw, Pb, chunk=2000):
    N = right.shape[0]
    W_aug = jnp.concatenate(
        [p['fin_W'], p['fin_b'][None, :], jnp.zeros((AUG - EMB - 1, EMB), F32)], axis=0)
    o1a = p['o1_W'][:EMB]
    o1b = p['o1_W'][EMB:]
    grid = (N // chunk,)
    bs_hs = pl.BlockSpec((chunk, AUG), lambda i: (i, 0))
    bs_r = pl.BlockSpec((chunk, EMB), lambda i: (i, 0))
    bs_wa = pl.BlockSpec((AUG, EMB), lambda i: (0, 0))
    bs_e = pl.BlockSpec((1, EMB), lambda i: (0, 0))
    bs_w = pl.BlockSpec((EMB, EMB), lambda i: (0, 0))
    bs_wp = pl.BlockSpec((EMB, AUG), lambda i: (0, 0))
    bs_ep = pl.BlockSpec((1, AUG), lambda i: (0, 0))
    bs_o = pl.BlockSpec((chunk, EMB), lambda i: (i, 0))
    bs_op = pl.BlockSpec((chunk, AUG), lambda i: (i, 0))
    bs_sum = pl.BlockSpec((1, EMB), lambda i: (0, 0))
    return pl.pallas_call(
        _post_kernel,
        grid=grid,
        in_specs=[bs_hs, bs_r, bs_wa, bs_e, bs_e, bs_w, bs_w, bs_e, bs_w,
                  bs_e, bs_wp, bs_ep],
        out_specs=[bs_o, bs_op, bs_sum],
        out_shape=[jax.ShapeDtypeStruct((N, EMB), F32),
                   jax.ShapeDtypeStruct((N, AUG), F32),
                   jax.ShapeDtypeStruct((1, EMB), F32)],
    )(hs_aug, right, W_aug, p['post_g'][None, :], p['post_b'][None, :],
      o1a, o1b, p['o1_b'][None, :], p['o2_W'], p['o2_b'][None, :],
      jnp.pad(Pw, ((0, 0), (0, AUG - EMB))),
      jnp.pad(Pb, (0, AUG - EMB))[None, :])


def _small_edge_kernel(a_ref, b_ref, src_ref, dst_ref, ev_ref, w0_ref,
                       g_ref, bln_ref, hs_ref):
    C = src_ref.shape[0]
    io = lax.broadcasted_iota(jnp.int32, (C, SROWS), 1)
    oh_src = (src_ref[...] == io).astype(F32)
    oh_dst = (dst_ref[...] == io).astype(F32)
    h = (jnp.dot(oh_dst, a_ref[...], preferred_element_type=F32)
         + jnp.dot(oh_src, b_ref[...], preferred_element_type=F32)
         + ev_ref[...] * w0_ref[...])
    h = _relu(_ln_rows(h, g_ref[...], bln_ref[...]))
    aug = jnp.concatenate(
        [h, jnp.ones((C, 1), F32), jnp.zeros((C, AUG - EMB - 1), F32)], axis=-1)
    acc = lax.dot_general(oh_dst, aug, (((0,), (0,)), ((), ())),
                          preferred_element_type=F32)

    @pl.when(pl.program_id(0) == 0)
    def _init():
        hs_ref[...] = jnp.zeros_like(hs_ref)
    hs_ref[...] += acc


def _small_edge(A32, B32, src, dst, ev, w0p, ln_g, ln_b, chunk=3400):
    E = src.shape[0]
    grid = (E // chunk,)
    bs_t = pl.BlockSpec((SROWS, EMB), lambda i: (0, 0))
    bs_i = pl.BlockSpec((chunk, 1), lambda i: (i, 0))
    bs_row = pl.BlockSpec((1, EMB), lambda i: (0, 0))
    bs_hs = pl.BlockSpec((SROWS, AUG), lambda i: (0, 0))
    return pl.pallas_call(
        _small_edge_kernel,
        grid=grid,
        in_specs=[bs_t, bs_t, bs_i, bs_i, bs_i, bs_row, bs_row, bs_row],
        out_specs=bs_hs,
        out_shape=jax.ShapeDtypeStruct((SROWS, AUG), F32),
    )(A32, B32, src, dst, ev, w0p, ln_g[None, :], ln_b[None, :])


def _small_post_kernel(hs_ref, right_ref, wa_ref, pg_ref, pb_ref,
                       o1a_ref, o1b_ref, o1bias_ref, o2w_ref, o2b_ref,
                       aw_ref, ab_ref, bw_ref, new_ref, a32_ref, b32_ref):
    agg = jnp.dot(hs_ref[...], wa_ref[...], preferred_element_type=F32)
    ln = _ln_rows(agg, pg_ref[...], pb_ref[...])
    t = _relu(jnp.dot(ln, o1a_ref[...], preferred_element_type=F32)
              + jnp.dot(right_ref[...], o1b_ref[...], preferred_element_type=F32)
              + o1bias_ref[...])
    new = jnp.dot(t, o2w_ref[...], preferred_element_type=F32) + o2b_ref[...]
    new_ref[...] = new
    a32_ref[...] = jnp.dot(new, aw_ref[...], preferred_element_type=F32) + ab_ref[...]
    b32_ref[...] = jnp.dot(new, bw_ref[...], preferred_element_type=F32)


def _small_post(hs32, right32, p, Aw, Ab, Bw):
    W_aug = jnp.concatenate(
        [p['fin_W'], p['fin_b'][None, :], jnp.zeros((AUG - EMB - 1, EMB), F32)], axis=0)
    sh = jax.ShapeDtypeStruct((SROWS, EMB), F32)
    return pl.pallas_call(
        _small_post_kernel,
        out_shape=[sh, sh, sh],
    )(hs32, right32, W_aug, p['post_g'][None, :], p['post_b'][None, :],
      p['o1_W'][:EMB], p['o1_W'][EMB:], p['o1_b'][None, :], p['o2_W'],
      p['o2_b'][None, :], Aw, Ab[None, :], Bw)


def _tconv_kernel(x_ref, xs_ref, src_ref, dst_ref, ev_ref,
                  qw_ref, qb_ref, kw_ref, kb_ref, vw_ref, vb_ref, te_ref,
                  skw_ref, skb_ref, sowx_ref, sowa_ref, sows_ref, sob_ref,
                  out_ref):
    x = x_ref[...]
    C = src_ref.shape[0]
    io = lax.broadcasted_iota(jnp.int32, (C, SROWS), 1)
    oh_src = (src_ref[...] == io).astype(F32)
    oh_dst = (dst_ref[...] == io).astype(F32)
    q = jnp.dot(x, qw_ref[...], preferred_element_type=F32) + qb_ref[...]
    k = jnp.dot(x, kw_ref[...], preferred_element_type=F32) + kb_ref[...]
    v = jnp.dot(x, vw_ref[...], preferred_element_type=F32) + vb_ref[...]
    e = ev_ref[...] * te_ref[...]
    kj = jnp.dot(oh_src, k, preferred_element_type=F32) + e
    qd = jnp.dot(oh_dst, q, preferred_element_type=F32)
    vj = jnp.dot(oh_src, v, preferred_element_type=F32) + e
    att = jnp.zeros((SROWS, EMB), F32)
    pad = ev_ref[...] * 0.0
    for h in range(4):
        sl = slice(h * 16, (h + 1) * 16)
        alpha = jnp.sum(qd[:, sl] * kj[:, sl], axis=-1, keepdims=True) * 0.25
        big = jnp.where(oh_dst > 0.0, alpha + pad, -1e30)
        amax = jnp.max(big, axis=0, keepdims=True)
        asub = jnp.sum(oh_dst * amax, axis=1, keepdims=True)
        ex = jnp.exp(alpha - asub)
        den = lax.dot_general(oh_dst, ex, (((0,), (0,)), ((), ())),
                              preferred_element_type=F32)
        dend = jnp.sum(oh_dst * den.T, axis=1, keepdims=True) + 1e-16
        a = ex / dend
        outh = vj[:, sl] * a
        aggh = lax.dot_general(oh_dst, outh, (((0,), (0,)), ((), ())),
                               preferred_element_type=F32)
        att = att + jnp.pad(aggh, ((0, 0), (h * 16, EMB - (h + 1) * 16)))
    att = att + jnp.dot(x, skw_ref[...], preferred_element_type=F32) + skb_ref[...]
    satt = _relu(jnp.dot(x, sowx_ref[...], preferred_element_type=F32)
                 + jnp.dot(att, sowa_ref[...], preferred_element_type=F32)
                 + xs_ref[...] * sows_ref[...] + sob_ref[...])
    mask = (lax.broadcasted_iota(jnp.int32, (SROWS, 1), 0) < 17).astype(F32)
    out_ref[...] = jnp.sum(satt * mask, axis=0, keepdims=True) * (1.0 / 17.0)


def _tconv_satt_mean(sep32, xs32, src, dst, ev, p):
    return pl.pallas_call(
        _tconv_kernel,
        out_shape=jax.ShapeDtypeStruct((1, EMB), F32),
    )(sep32, xs32, src, dst, ev,
      p['tq_W'], p['tq_b'][None, :], p['tk_W'], p['tk_b'][None, :],
      p['tv_W'], p['tv_b'][None, :], p['te_W'][0][None, :],
      p['tskip_W'], p['tskip_b'][None, :],
      p['so_W'][:EMB], p['so_W'][EMB:2 * EMB], p['so_W'][2 * EMB][None, :],
      p['so_b'][None, :])


def _final_kernel(sa_ref, ra_ref, ca_ref, w1a_ref, w1b_ref, w1c_ref,
                  b1_ref, w2_ref, b2_ref, out_ref):
    h = _relu(jnp.dot(sa_ref[...], w1a_ref[...], preferred_element_type=F32)
              + jnp.dot(ra_ref[...], w1b_ref[...], preferred_element_type=F32)
              + jnp.dot(ca_ref[...], w1c_ref[...], preferred_element_type=F32)
              + b1_ref[...])
    z = jnp.dot(h, w2_ref[...], preferred_element_type=F32) + b2_ref[...]
    out_ref[...] = 1.0 / (1.0 + jnp.exp(-z))


def _final(sa, ra, ca, p):
    return pl.pallas_call(
        _final_kernel,
        out_shape=jax.ShapeDtypeStruct((1, 1), F32),
    )(sa, ra, ca, p['out_W1'][:EMB], p['out_W1'][EMB:2 * EMB],
      p['out_W1'][2 * EMB:], p['out_b1'][None, :], p['out_W2'],
      p['out_b2'][None, :])


def _proj_kernel(x_ref, w_ref, b_ref, o_ref):
    o_ref[...] = (jnp.dot(x_ref[...], w_ref[...], preferred_element_type=F32)
                  + b_ref[...])


def _proj32(x32, W, bvec):
    return pl.pallas_call(
        _proj_kernel, out_shape=jax.ShapeDtypeStruct((SROWS, EMB), F32),
    )(x32, W, bvec[None, :])


def _rattsum_kernel(x_ref, w_ref, b_ref, o_ref):
    @pl.when(pl.program_id(0) == 0)
    def _init():
        o_ref[...] = jnp.zeros_like(o_ref)
    o_ref[...] += jnp.sum(
        _relu(jnp.dot(x_ref[...], w_ref[...], preferred_element_type=F32)
              + b_ref[...]), axis=0, keepdims=True)


def _relu_matsum(x, W, bvec, chunk=2000):
    N = x.shape[0]
    return pl.pallas_call(
        _rattsum_kernel,
        grid=(N // chunk,),
        in_specs=[pl.BlockSpec((chunk, EMB), lambda i: (i, 0)),
                  pl.BlockSpec((EMB, EMB), lambda i: (0, 0)),
                  pl.BlockSpec((1, EMB), lambda i: (0, 0))],
        out_specs=pl.BlockSpec((1, EMB), lambda i: (0, 0)),
        out_shape=jax.ShapeDtypeStruct((1, EMB), F32),
    )(x, W, bvec[None, :])


def _edge_fold(ev_2xE_pad, count, g, b, e_W):
    st = [_col_stats(ev_2xE_pad[d]) for d in range(ev_2xE_pad.shape[0])]
    s_list, c_parts = [], []
    for d, std in enumerate(st):
        tot = jnp.sum(std[0])
        totq = jnp.sum(std[1])
        m = tot / count
        v = totq / count - m * m
        s = g[d] * lax.rsqrt(v + 1e-5)
        s_list.append(s)
        c_parts.append((b[d] - m * s) * e_W[d])
    Wp = jnp.stack([e_W[d] * s_list[d] for d in range(len(st))])
    c = sum(c_parts)
    return Wp, c


_GDN = lax.GatherDimensionNumbers(offset_dims=(), collapsed_slice_dims=(0,),
                                  start_index_map=(0,))


def _lanesum(v):
    io = lax.iota(jnp.int32, 16)
    for k in (8, 4, 2, 1):
        g = lax.gather(v, (io ^ k)[:, None], _GDN, (1,),
                       mode=lax.GatherScatterMode.PROMISE_IN_BOUNDS)
        v = v + g
    return v


ROWS_HALF = 25000
ROWS_PC = 25088
SC_CH = 256
SC_NCH = 196
E_PAD = 16 * SC_NCH * SC_CH


def _sc_gather_body(a_h, b_h, src_h, dst_h, out_h,
                    si, di, ar, br, ob, sem_a, sem_b, sem_o):
    core = lax.axis_index("c")
    sid = lax.axis_index("s")
    wid = sid * 2 + core

    def _chunk(c, cr):
        base = wid * (SC_NCH_G * SC_CH) + c * SC_CH
        pltpu.sync_copy(src_h.at[pl.ds(base, SC_CH)], si)
        pltpu.sync_copy(dst_h.at[pl.ds(base, SC_CH)], di)
        cpa = pltpu.async_copy(a_h.at[di], ar, sem_a)
        cpb = pltpu.async_copy(b_h.at[si], br, sem_b)
        cpa.wait()
        cpb.wait()

        def _egroup(g, ce):
            gbase = g * 16
            for j in range(16):
                e = gbase + j
                for k in range(4):
                    av = ar[e, pl.ds(16 * k, 16)]
                    bv = br[e, pl.ds(16 * k, 16)]
                    ob[e, pl.ds(16 * k, 16)] = av + bv
            return ce
        lax.fori_loop(0, SC_CH // 16, _egroup, 0)
        pltpu.async_copy(ob, out_h.at[pl.ds(base, SC_CH)], sem_o).wait()
        return cr
    lax.fori_loop(0, SC_NCH_G, _chunk, 0)


def _sc_scatter_body(h_h, dst_h, out_h, di, mi, hb, zb, acc, sem_h):
    core = lax.axis_index("c")
    sid = lax.axis_index("s")
    io16 = lax.iota(jnp.int32, 16)
    zero16 = jnp.zeros((16,), F32)

    def _zrow(r, cr):
        for t in range(AUG // 16):
            zb[r, pl.ds(16 * t, 16)] = zero16
        return cr
    lax.fori_loop(0, SC_CH, _zrow, 0)

    for ps in range(4):
        q = core * 4 + ps
        live = q < NPASS
        lo = q * QROWS
        zbase = sid * (ACC_ROWS // 16)

        @pl.when(live)
        def _zero():
            pltpu.sync_copy(zb.at[pl.ds(0, 248)],
                            acc.at[pl.ds(zbase, 248)])
            pltpu.sync_copy(zb.at[pl.ds(0, 248)],
                            acc.at[pl.ds(zbase + 248, 248)])
            pltpu.sync_copy(zb.at[pl.ds(0, 8)],
                            acc.at[pl.ds(zbase + 496, 8)])
        plsc.subcore_barrier()

        def _chunk(c, cr):
            base = sid * (SC_NCH_S * SC_CH) + c * SC_CH
            pltpu.sync_copy(dst_h.at[pl.ds(base, SC_CH)], di)
            cph = pltpu.async_copy(h_h.at[pl.ds(base, SC_CH)], hb, sem_h)

            def _gmap(g, cg):
                d = di[pl.ds(g * 16, 16)]
                local = d - lo
                eg = io16 + (base + g * 16)
                ok = (local >= 0) & (local < QROWS) & (eg < E_RC_TOTAL)
                trash = QROWS + (io16 & 7)
                mi[pl.ds(g * 16, 16)] = jnp.where(ok, local, trash)
                return cg
            lax.fori_loop(0, SC_CH // 16, _gmap, 0)
            cph.wait()
            pltpu.sync_copy(hb, acc.at[mi], add=True)
            return cr

        @pl.when(live)
        def _scan():
            lax.fori_loop(0, SC_NCH_S, _chunk, 0)
        plsc.subcore_barrier()

        @pl.when(live & (sid == 0))
        def _flush():
            pltpu.sync_copy(acc.at[pl.ds(0, QROWS)], out_h.at[q])
        plsc.subcore_barrier()


E_RC_TOTAL = 800000
SC_NCH_G = E_PAD // (32 * SC_CH)
SC_NCH_S = E_PAD // (16 * SC_CH)
QROWS = 8000
NPASS = 7
ACC_ROWS = 8064


def _ln_aug_kernel(h_ref, e0_ref, e1_ref, w0_ref, w1_ref, g_ref, b_ref,
                   o_ref):
    h = (h_ref[:, :EMB] + e0_ref[...] * w0_ref[...]
         + e1_ref[...] * w1_ref[...])
    ln = _relu(_ln_rows(h, g_ref[...], b_ref[...]))
    C = ln.shape[0]
    o_ref[...] = jnp.concatenate(
        [ln, jnp.ones((C, 1), F32), jnp.zeros((C, AUG - EMB - 1), F32)],
        axis=-1)


def _sc_edge_stage(A, B, srcp, dstp, ev0p, ev1p, w0p, w1p, ln_g, ln_b):
    mesh = plsc.VectorSubcoreMesh(core_axis_name="c", subcore_axis_name="s")

    gather = functools.partial(
        pl.kernel, mesh=mesh,
        out_type=jax.ShapeDtypeStruct((E_PAD, AUG), F32),
        scratch_types=[
            pltpu.VMEM((SC_CH,), jnp.int32),
            pltpu.VMEM((SC_CH,), jnp.int32),
            pltpu.VMEM((SC_CH, AUG), F32),
            pltpu.VMEM((SC_CH, AUG), F32),
            pltpu.VMEM((SC_CH, AUG), F32),
            pltpu.SemaphoreType.DMA,
            pltpu.SemaphoreType.DMA,
            pltpu.SemaphoreType.DMA,
        ])(_sc_gather_body)
    h = gather(A, B, srcp, dstp)

    CH2 = 2048
    haug = pl.pallas_call(
        _ln_aug_kernel,
        grid=(E_PAD // CH2,),
        in_specs=[pl.BlockSpec((CH2, AUG), lambda i: (i, 0)),
                  pl.BlockSpec((CH2, 1), lambda i: (i, 0)),
                  pl.BlockSpec((CH2, 1), lambda i: (i, 0)),
                  pl.BlockSpec((1, EMB), lambda i: (0, 0)),
                  pl.BlockSpec((1, EMB), lambda i: (0, 0)),
                  pl.BlockSpec((1, EMB), lambda i: (0, 0)),
                  pl.BlockSpec((1, EMB), lambda i: (0, 0))],
        out_specs=pl.BlockSpec((CH2, AUG), lambda i: (i, 0)),
        out_shape=jax.ShapeDtypeStruct((E_PAD, AUG), F32),
    )(h, ev0p[:, None], ev1p[:, None], w0p[None, :], w1p[None, :],
      ln_g[None, :], ln_b[None, :])

    scatter = functools.partial(
        pl.kernel, mesh=mesh,
        out_type=jax.ShapeDtypeStruct((NPASS, QROWS, AUG), F32),
        scratch_types=[
            pltpu.VMEM((SC_CH,), jnp.int32),
            pltpu.VMEM((SC_CH,), jnp.int32),
            pltpu.VMEM((SC_CH, AUG), F32),
            pltpu.VMEM((SC_CH, AUG), F32),
            pltpu.VMEM_SHARED((ACC_ROWS, AUG), F32),
            pltpu.SemaphoreType.DMA,
        ])(_sc_scatter_body)
    out = scatter(haug, dstp)
    return out.reshape(NPASS * QROWS, AUG)[:50000]


def kernel(x_rows, x_cols, x_sepas, edge_index_rowcols, edge_vals_rowcols,
           edge_index_sepa_cols, edge_vals_sepa_cols, edge_index_sepa_rows,
           edge_vals_sepa_rows, edge_index_sepa_self, edge_vals_sepa_self,
           params):
    p = params
    ei_rc = edge_index_rowcols.astype(jnp.int32)
    ei_sc = edge_index_sepa_cols.astype(jnp.int32)
    ei_sr = edge_index_sepa_rows.astype(jnp.int32)
    ei_ss = edge_index_sepa_self.astype(jnp.int32)

    E_RC = ei_rc.shape[1]
    E_SC = ei_sc.shape[1]
    E_SR = ei_sr.shape[1]

    evT_rc = edge_vals_rowcols.T.reshape(2, E_RC // 128, 128)
    Wp_rc, c_rc = _edge_fold(evT_rc, E_RC, p['en_rowcols_g'],
                             p['en_rowcols_b'], p['c2r']['e_W'])
    Wp_rc2, c_rc2 = _edge_fold(evT_rc, E_RC, p['en_rowcols_g'],
                               p['en_rowcols_b'], p['r2c']['e_W'])

    def _pad128(v):
        E = v.shape[0]
        R = -(-E // 128) * 128
        return jnp.pad(v, (0, R - E)).reshape(1, R // 128, 128)

    Wp_sc, c_sc = _edge_fold(_pad128(edge_vals_sepa_cols[:, 0]), E_SC,
                             p['en_sepas_g'], p['en_sepas_b'], p['c2s']['e_W'])
    Wp_sr, c_sr = _edge_fold(_pad128(edge_vals_sepa_rows[:, 0]), E_SR,
                             p['en_rows_g'], p['en_rows_b'], p['s2r']['e_W'])
    Wp_r2s, c_r2s = _edge_fold(_pad128(edge_vals_sepa_rows[:, 0]), E_SR,
                               p['en_rows_g'], p['en_rows_b'], p['r2s']['e_W'])

    row0, A_c2r, _ = _prologue(
        x_rows, p['row_bn_g'], p['row_bn_b'], p['row_W1'], p['row_b1'],
        p['row_W2'], p['row_b2'],
        p['c2r']['l_W'], p['c2r']['l_b'] + c_rc,
        jnp.zeros((EMB, EMB), F32), jnp.zeros((EMB,), F32))
    col0, B_c2r, A_r2c = _prologue(
        x_cols, p['col_bn_g'], p['col_bn_b'], p['col_W1'], p['col_b1'],
        p['col_W2'], p['col_b2'],
        p['c2r']['r_W'], jnp.zeros((EMB,), F32),
        p['r2c']['l_W'], p['r2c']['l_b'] + c_rc2)

    padn = E_PAD - E_RC
    srcp = jnp.pad(ei_rc[1], (0, padn))
    dstp = jnp.pad(ei_rc[0], (0, padn))
    ev0p = jnp.pad(edge_vals_rowcols[:, 0], (0, padn))
    ev1p = jnp.pad(edge_vals_rowcols[:, 1], (0, padn))
    hs_c2r = _sc_edge_stage(A_c2r, B_c2r, srcp, dstp, ev0p, ev1p,
                            Wp_rc[0], Wp_rc[1],
                            p['c2r']['fin_g'], p['c2r']['fin_bln'])
    row1, B_r2c, _ = _post(hs_c2r, row0, p['c2r'],
                           p['r2c']['r_W'], jnp.zeros((EMB,), F32))

    hs_r2c = _sc_edge_stage(A_r2c, B_r2c, dstp, srcp, ev0p, ev1p,
                            Wp_rc2[0], Wp_rc2[1],
                            p['r2c']['fin_g'], p['r2c']['fin_bln'])
    col1, _, colsum = _post(hs_r2c, col0, p['r2c'],
                            jnp.zeros((EMB, EMB), F32), jnp.zeros((EMB,), F32))

    sep_b = p['sepa_ln_b'][None, :]
    sep0_row = _relu(_relu(sep_b @ p['sepa_W1'] + p['sepa_b1'])
                     @ p['sepa_W2'] + p['sepa_b2'])
    sep0 = jnp.broadcast_to(sep0_row, (SROWS, EMB)) * (
        (jnp.arange(SROWS) < 17).astype(F32)[:, None])
    A32_c2s = _proj32(sep0, p['c2s']['l_W'], p['c2s']['l_b'] + c_sc)
    B32_c2s = _proj32(jnp.pad(col1[:17], ((0, SROWS - 17), (0, 0))),
                      p['c2s']['r_W'], jnp.zeros((EMB,), F32))

    src_sc = ei_sc[1][:, None]
    dst_sc = ei_sc[0][:, None]
    hs_c2s = _small_edge(A32_c2s, B32_c2s, src_sc, dst_sc,
                         edge_vals_sepa_cols, Wp_sc[0][None, :],
                         p['c2s']['fin_g'], p['c2s']['fin_bln'])
    sep1, _, B32_s2r = _small_post(
        hs_c2s, sep0, p['c2s'],
        jnp.zeros((EMB, EMB), F32), jnp.zeros((EMB,), F32), p['s2r']['r_W'])
    row1_17 = jnp.pad(row1[:17], ((0, SROWS - 17), (0, 0)))
    A32_s2r = _proj32(row1_17, p['s2r']['l_W'], p['s2r']['l_b'] + c_sr)

    src_sr = ei_sr[0][:, None]
    dst_sr = ei_sr[1][:, None]
    hs_s2r17 = _small_edge(A32_s2r, B32_s2r, src_sr, dst_sr,
                           edge_vals_sepa_rows, Wp_sr[0][None, :],
                           p['s2r']['fin_g'], p['s2r']['fin_bln'])
    hs_s2r = jnp.concatenate(
        [hs_s2r17[:17], jnp.zeros((row1.shape[0] - 17, AUG), F32)], axis=0)
    row2, _, _ = _post(hs_s2r, row1, p['s2r'],
                       jnp.zeros((EMB, EMB), F32), jnp.zeros((EMB,), F32))

    row2_17 = jnp.pad(row2[:17], ((0, SROWS - 17), (0, 0)))
    A32_r2s = _proj32(sep1, p['r2s']['l_W'], p['r2s']['l_b'] + c_r2s)
    B32_r2s = _proj32(row2_17, p['r2s']['r_W'], jnp.zeros((EMB,), F32))

    hs_r2s = _small_edge(A32_r2s, B32_r2s, dst_sr, src_sr,
                         edge_vals_sepa_rows, Wp_r2s[0][None, :],
                         p['r2s']['fin_g'], p['r2s']['fin_bln'])
    sep2, _, _ = _small_post(hs_r2s, sep1, p['r2s'],
                             jnp.zeros((EMB, EMB), F32),
                             jnp.zeros((EMB,), F32),
                             jnp.zeros((EMB, EMB), F32))

    E_SS = ei_ss.shape[1]
    PSS = -(-E_SS // 8) * 8
    src_ss = jnp.pad(ei_ss[0], (0, PSS - E_SS),
                     constant_values=SROWS - 1)[:, None]
    dst_ss = jnp.pad(ei_ss[1], (0, PSS - E_SS),
                     constant_values=SROWS - 1)[:, None]
    ev_ss = jnp.pad(edge_vals_sepa_self[:, 0], (0, PSS - E_SS))[:, None]
    xs32 = jnp.pad(x_sepas, ((0, SROWS - 17), (0, 0)))
    sattmean = _tconv_satt_mean(sep2, xs32, src_ss, dst_ss, ev_ss, p)

    N = row2.shape[0]
    rattsum = _relu_matsum(row2, p['ro_W'], p['ro_b'])

    return _final(sattmean, rattsum / N, colsum / N, p)


# speedup vs baseline: 2.2971x; 1.5680x over previous
import functools

import jax
import jax.numpy as jnp
from jax import lax
from jax.experimental import pallas as pl
from jax.experimental.pallas import tpu as pltpu
from jax.experimental.pallas import tpu_sc as plsc

EMB = 64
AUG = 128
SROWS = 32
F32 = jnp.float32


def _relu(x):
    return jnp.maximum(x, 0.0)


def _ln_rows(x, g, b):
    m = jnp.mean(x, axis=-1, keepdims=True)
    v = jnp.mean(x * x, axis=-1, keepdims=True) - m * m
    return (x - m) * lax.rsqrt(v + 1e-5) * g + b


def _stats_kernel(x_ref, o_ref):
    x = x_ref[...]
    s = jnp.sum(x, axis=0, keepdims=True)
    q = jnp.sum(x * x, axis=0, keepdims=True)
    o_ref[...] = jnp.concatenate([s, q], axis=0)


def _col_stats(x):
    return pl.pallas_call(
        _stats_kernel,
        out_shape=jax.ShapeDtypeStruct((2, x.shape[1]), F32),
    )(x)


def _prologue_kernel(x_ref, m_ref, s_ref, bb_ref, w1_ref, b1_ref, w2_ref,
                     b2_ref, p1w_ref, p1b_ref, p2w_ref, p2b_ref,
                     emb_ref, p1_ref, p2_ref):
    x = x_ref[...]
    xn = (x - m_ref[...]) * s_ref[...] + bb_ref[...]
    h = _relu(jnp.dot(xn, w1_ref[...], preferred_element_type=F32) + b1_ref[...])
    h = _relu(jnp.dot(h, w2_ref[...], preferred_element_type=F32) + b2_ref[...])
    emb_ref[...] = h
    p1_ref[...] = jnp.dot(h, p1w_ref[...], preferred_element_type=F32) + p1b_ref[...]
    p2_ref[...] = jnp.dot(h, p2w_ref[...], preferred_element_type=F32) + p2b_ref[...]


def _prologue(x, bn_g, bn_b, W1, b1, W2, b2, P1w, P1b, P2w, P2b, chunk=2000):
    N, D = x.shape
    st = _col_stats(x)
    m = st[0] / N
    var = st[1] / N - m * m
    scale = lax.rsqrt(var + 1e-5) * bn_g
    grid = (N // chunk,)
    bs_x = pl.BlockSpec((chunk, D), lambda i: (i, 0))
    bs_row = pl.BlockSpec((1, D), lambda i: (0, 0))
    bs_w1 = pl.BlockSpec((D, EMB), lambda i: (0, 0))
    bs_e = pl.BlockSpec((1, EMB), lambda i: (0, 0))
    bs_w = pl.BlockSpec((EMB, EMB), lambda i: (0, 0))
    bs_wp = pl.BlockSpec((EMB, AUG), lambda i: (0, 0))
    bs_ep = pl.BlockSpec((1, AUG), lambda i: (0, 0))
    bs_o = pl.BlockSpec((chunk, EMB), lambda i: (i, 0))
    bs_op = pl.BlockSpec((chunk, AUG), lambda i: (i, 0))
    out_sh = jax.ShapeDtypeStruct((N, EMB), F32)
    out_shp = jax.ShapeDtypeStruct((N, AUG), F32)
    padw = lambda W: jnp.pad(W, ((0, 0), (0, AUG - EMB)))
    padb = lambda b: jnp.pad(b, (0, AUG - EMB))
    return pl.pallas_call(
        _prologue_kernel,
        grid=grid,
        in_specs=[bs_x, bs_row, bs_row, bs_row, bs_w1, bs_e, bs_w, bs_e,
                  bs_wp, bs_ep, bs_wp, bs_ep],
        out_specs=[bs_o, bs_op, bs_op],
        out_shape=[out_sh, out_shp, out_shp],
    )(x, m[None, :], scale[None, :], bn_b[None, :], W1, b1[None, :], W2,
      b2[None, :], padw(P1w), padb(P1b)[None, :], padw(P2w),
      padb(P2b)[None, :])


def _post_kernel(hs_ref, right_ref, wa_ref, pg_ref, pb_ref,
                 o1a_ref, o1b_ref, o1bias_ref, o2w_ref, o2b_ref,
                 pw_ref, pbias_ref, new_ref, proj_ref, sum_ref):
    agg = jnp.dot(hs_ref[...], wa_ref[...], preferred_element_type=F32)
    ln = _ln_rows(agg, pg_ref[...], pb_ref[...])
    t = (jnp.dot(ln, o1a_ref[...], preferred_element_type=F32)
         + jnp.dot(right_ref[...], o1b_ref[...], preferred_element_type=F32)
         + o1bias_ref[...])
    t = _relu(t)
    new = jnp.dot(t, o2w_ref[...], preferred_element_type=F32) + o2b_ref[...]
    new_ref[...] = new
    proj_ref[...] = jnp.dot(new, pw_ref[...], preferred_element_type=F32) + pbias_ref[...]

    @pl.when(pl.program_id(0) == 0)
    def _init():
        sum_ref[...] = jnp.zeros_like(sum_ref)
    sum_ref[...] += jnp.sum(new, axis=0, keepdims=True)


def _post(hs_aug, right, p, Pw, Pb, chunk=2000):
    N = right.shape[0]
    W_aug = jnp.concatenate(
        [p['fin_W'], p['fin_b'][None, :], jnp.zeros((AUG - EMB - 1, EMB), F32)], axis=0)
    o1a = p['o1_W'][:EMB]
    o1b = p['o1_W'][EMB:]
    grid = (N // chunk,)
    bs_hs = pl.BlockSpec((chunk, AUG), lambda i: (i, 0))
    bs_r = pl.BlockSpec((chunk, EMB), lambda i: (i, 0))
    bs_wa = pl.BlockSpec((AUG, EMB), lambda i: (0, 0))
    bs_e = pl.BlockSpec((1, EMB), lambda i: (0, 0))
    bs_w = pl.BlockSpec((EMB, EMB), lambda i: (0, 0))
    bs_wp = pl.BlockSpec((EMB, AUG), lambda i: (0, 0))
    bs_ep = pl.BlockSpec((1, AUG), lambda i: (0, 0))
    bs_o = pl.BlockSpec((chunk, EMB), lambda i: (i, 0))
    bs_op = pl.BlockSpec((chunk, AUG), lambda i: (i, 0))
    bs_sum = pl.BlockSpec((1, EMB), lambda i: (0, 0))
    return pl.pallas_call(
        _post_kernel,
        grid=grid,
        in_specs=[bs_hs, bs_r, bs_wa, bs_e, bs_e, bs_w, bs_w, bs_e, bs_w,
                  bs_e, bs_wp, bs_ep],
        out_specs=[bs_o, bs_op, bs_sum],
        out_shape=[jax.ShapeDtypeStruct((N, EMB), F32),
                   jax.ShapeDtypeStruct((N, AUG), F32),
                   jax.ShapeDtypeStruct((1, EMB), F32)],
    )(hs_aug, right, W_aug, p['post_g'][None, :], p['post_b'][None, :],
      o1a, o1b, p['o1_b'][None, :], p['o2_W'], p['o2_b'][None, :],
      jnp.pad(Pw, ((0, 0), (0, AUG - EMB))),
      jnp.pad(Pb, (0, AUG - EMB))[None, :])


def _small_edge_kernel(a_ref, b_ref, src_ref, dst_ref, ev_ref, w0_ref,
                       g_ref, bln_ref, hs_ref):
    C = src_ref.shape[0]
    io = lax.broadcasted_iota(jnp.int32, (C, SROWS), 1)
    oh_src = (src_ref[...] == io).astype(F32)
    oh_dst = (dst_ref[...] == io).astype(F32)
    h = (jnp.dot(oh_dst, a_ref[...], preferred_element_type=F32)
         + jnp.dot(oh_src, b_ref[...], preferred_element_type=F32)
         + ev_ref[...] * w0_ref[...])
    h = _relu(_ln_rows(h, g_ref[...], bln_ref[...]))
    aug = jnp.concatenate(
        [h, jnp.ones((C, 1), F32), jnp.zeros((C, AUG - EMB - 1), F32)], axis=-1)
    acc = lax.dot_general(oh_dst, aug, (((0,), (0,)), ((), ())),
                          preferred_element_type=F32)

    @pl.when(pl.program_id(0) == 0)
    def _init():
        hs_ref[...] = jnp.zeros_like(hs_ref)
    hs_ref[...] += acc


def _small_edge(A32, B32, src, dst, ev, w0p, ln_g, ln_b, chunk=3400):
    E = src.shape[0]
    grid = (E // chunk,)
    bs_t = pl.BlockSpec((SROWS, EMB), lambda i: (0, 0))
    bs_i = pl.BlockSpec((chunk, 1), lambda i: (i, 0))
    bs_row = pl.BlockSpec((1, EMB), lambda i: (0, 0))
    bs_hs = pl.BlockSpec((SROWS, AUG), lambda i: (0, 0))
    return pl.pallas_call(
        _small_edge_kernel,
        grid=grid,
        in_specs=[bs_t, bs_t, bs_i, bs_i, bs_i, bs_row, bs_row, bs_row],
        out_specs=bs_hs,
        out_shape=jax.ShapeDtypeStruct((SROWS, AUG), F32),
    )(A32, B32, src, dst, ev, w0p, ln_g[None, :], ln_b[None, :])


def _small_post_kernel(hs_ref, right_ref, wa_ref, pg_ref, pb_ref,
                       o1a_ref, o1b_ref, o1bias_ref, o2w_ref, o2b_ref,
                       aw_ref, ab_ref, bw_ref, new_ref, a32_ref, b32_ref):
    agg = jnp.dot(hs_ref[...], wa_ref[...], preferred_element_type=F32)
    ln = _ln_rows(agg, pg_ref[...], pb_ref[...])
    t = _relu(jnp.dot(ln, o1a_ref[...], preferred_element_type=F32)
              + jnp.dot(right_ref[...], o1b_ref[...], preferred_element_type=F32)
              + o1bias_ref[...])
    new = jnp.dot(t, o2w_ref[...], preferred_element_type=F32) + o2b_ref[...]
    new_ref[...] = new
    a32_ref[...] = jnp.dot(new, aw_ref[...], preferred_element_type=F32) + ab_ref[...]
    b32_ref[...] = jnp.dot(new, bw_ref[...], preferred_element_type=F32)


def _small_post(hs32, right32, p, Aw, Ab, Bw):
    W_aug = jnp.concatenate(
        [p['fin_W'], p['fin_b'][None, :], jnp.zeros((AUG - EMB - 1, EMB), F32)], axis=0)
    sh = jax.ShapeDtypeStruct((SROWS, EMB), F32)
    return pl.pallas_call(
        _small_post_kernel,
        out_shape=[sh, sh, sh],
    )(hs32, right32, W_aug, p['post_g'][None, :], p['post_b'][None, :],
      p['o1_W'][:EMB], p['o1_W'][EMB:], p['o1_b'][None, :], p['o2_W'],
      p['o2_b'][None, :], Aw, Ab[None, :], Bw)


def _tconv_kernel(x_ref, xs_ref, src_ref, dst_ref, ev_ref,
                  qw_ref, qb_ref, kw_ref, kb_ref, vw_ref, vb_ref, te_ref,
                  skw_ref, skb_ref, sowx_ref, sowa_ref, sows_ref, sob_ref,
                  out_ref):
    x = x_ref[...]
    C = src_ref.shape[0]
    io = lax.broadcasted_iota(jnp.int32, (C, SROWS), 1)
    oh_src = (src_ref[...] == io).astype(F32)
    oh_dst = (dst_ref[...] == io).astype(F32)
    q = jnp.dot(x, qw_ref[...], preferred_element_type=F32) + qb_ref[...]
    k = jnp.dot(x, kw_ref[...], preferred_element_type=F32) + kb_ref[...]
    v = jnp.dot(x, vw_ref[...], preferred_element_type=F32) + vb_ref[...]
    e = ev_ref[...] * te_ref[...]
    kj = jnp.dot(oh_src, k, preferred_element_type=F32) + e
    qd = jnp.dot(oh_dst, q, preferred_element_type=F32)
    vj = jnp.dot(oh_src, v, preferred_element_type=F32) + e
    att = jnp.zeros((SROWS, EMB), F32)
    pad = ev_ref[...] * 0.0
    for h in range(4):
        sl = slice(h * 16, (h + 1) * 16)
        alpha = jnp.sum(qd[:, sl] * kj[:, sl], axis=-1, keepdims=True) * 0.25
        big = jnp.where(oh_dst > 0.0, alpha + pad, -1e30)
        amax = jnp.max(big, axis=0, keepdims=True)
        asub = jnp.sum(oh_dst * amax, axis=1, keepdims=True)
        ex = jnp.exp(alpha - asub)
        den = lax.dot_general(oh_dst, ex, (((0,), (0,)), ((), ())),
                              preferred_element_type=F32)
        dend = jnp.sum(oh_dst * den.T, axis=1, keepdims=True) + 1e-16
        a = ex / dend
        outh = vj[:, sl] * a
        aggh = lax.dot_general(oh_dst, outh, (((0,), (0,)), ((), ())),
                               preferred_element_type=F32)
        att = att + jnp.pad(aggh, ((0, 0), (h * 16, EMB - (h + 1) * 16)))
    att = att + jnp.dot(x, skw_ref[...], preferred_element_type=F32) + skb_ref[...]
    satt = _relu(jnp.dot(x, sowx_ref[...], preferred_element_type=F32)
                 + jnp.dot(att, sowa_ref[...], preferred_element_type=F32)
                 + xs_ref[...] * sows_ref[...] + sob_ref[...])
    mask = (lax.broadcasted_iota(jnp.int32, (SROWS, 1), 0) < 17).astype(F32)
    out_ref[...] = jnp.sum(satt * mask, axis=0, keepdims=True) * (1.0 / 17.0)


def _tconv_satt_mean(sep32, xs32, src, dst, ev, p):
    return pl.pallas_call(
        _tconv_kernel,
        out_shape=jax.ShapeDtypeStruct((1, EMB), F32),
    )(sep32, xs32, src, dst, ev,
      p['tq_W'], p['tq_b'][None, :], p['tk_W'], p['tk_b'][None, :],
      p['tv_W'], p['tv_b'][None, :], p['te_W'][0][None, :],
      p['tskip_W'], p['tskip_b'][None, :],
      p['so_W'][:EMB], p['so_W'][EMB:2 * EMB], p['so_W'][2 * EMB][None, :],
      p['so_b'][None, :])


def _final_kernel(sa_ref, ra_ref, ca_ref, w1a_ref, w1b_ref, w1c_ref,
                  b1_ref, w2_ref, b2_ref, out_ref):
    h = _relu(jnp.dot(sa_ref[...], w1a_ref[...], preferred_element_type=F32)
              + jnp.dot(ra_ref[...], w1b_ref[...], preferred_element_type=F32)
              + jnp.dot(ca_ref[...], w1c_ref[...], preferred_element_type=F32)
              + b1_ref[...])
    z = jnp.dot(h, w2_ref[...], preferred_element_type=F32) + b2_ref[...]
    out_ref[...] = 1.0 / (1.0 + jnp.exp(-z))


def _final(sa, ra, ca, p):
    return pl.pallas_call(
        _final_kernel,
        out_shape=jax.ShapeDtypeStruct((1, 1), F32),
    )(sa, ra, ca, p['out_W1'][:EMB], p['out_W1'][EMB:2 * EMB],
      p['out_W1'][2 * EMB:], p['out_b1'][None, :], p['out_W2'],
      p['out_b2'][None, :])


def _proj_kernel(x_ref, w_ref, b_ref, o_ref):
    o_ref[...] = (jnp.dot(x_ref[...], w_ref[...], preferred_element_type=F32)
                  + b_ref[...])


def _proj32(x32, W, bvec):
    return pl.pallas_call(
        _proj_kernel, out_shape=jax.ShapeDtypeStruct((SROWS, EMB), F32),
    )(x32, W, bvec[None, :])


def _rattsum_kernel(x_ref, w_ref, b_ref, o_ref):
    @pl.when(pl.program_id(0) == 0)
    def _init():
        o_ref[...] = jnp.zeros_like(o_ref)
    o_ref[...] += jnp.sum(
        _relu(jnp.dot(x_ref[...], w_ref[...], preferred_element_type=F32)
              + b_ref[...]), axis=0, keepdims=True)


def _relu_matsum(x, W, bvec, chunk=2000):
    N = x.shape[0]
    return pl.pallas_call(
        _rattsum_kernel,
        grid=(N // chunk,),
        in_specs=[pl.BlockSpec((chunk, EMB), lambda i: (i, 0)),
                  pl.BlockSpec((EMB, EMB), lambda i: (0, 0)),
                  pl.BlockSpec((1, EMB), lambda i: (0, 0))],
        out_specs=pl.BlockSpec((1, EMB), lambda i: (0, 0)),
        out_shape=jax.ShapeDtypeStruct((1, EMB), F32),
    )(x, W, bvec[None, :])


def _edge_fold(ev_2xE_pad, count, g, b, e_W):
    st = [_col_stats(ev_2xE_pad[d]) for d in range(ev_2xE_pad.shape[0])]
    s_list, c_parts = [], []
    for d, std in enumerate(st):
        tot = jnp.sum(std[0])
        totq = jnp.sum(std[1])
        m = tot / count
        v = totq / count - m * m
        s = g[d] * lax.rsqrt(v + 1e-5)
        s_list.append(s)
        c_parts.append((b[d] - m * s) * e_W[d])
    Wp = jnp.stack([e_W[d] * s_list[d] for d in range(len(st))])
    c = sum(c_parts)
    return Wp, c


SC_CH = 256
SC_CHS = 64
SC_NCH = 196
E_PAD = 16 * SC_NCH * SC_CH


def _sc_gather_body(a_h, b_h, src_h, dst_h, out_h,
                    si, di, ar, br, ob, sem_a, sem_b, sem_o):
    core = lax.axis_index("c")
    sid = lax.axis_index("s")
    wid = sid * 2 + core

    def _chunk(c, cr):
        base = wid * (SC_NCH_G * SC_CH) + c * SC_CH
        pltpu.sync_copy(src_h.at[pl.ds(base, SC_CH)], si)
        pltpu.sync_copy(dst_h.at[pl.ds(base, SC_CH)], di)
        cpa = pltpu.async_copy(a_h.at[di], ar, sem_a)
        cpb = pltpu.async_copy(b_h.at[si], br, sem_b)
        cpa.wait()
        cpb.wait()

        def _egroup(g, ce):
            gbase = g * 16
            for j in range(16):
                e = gbase + j
                for k in range(4):
                    av = ar[e, pl.ds(16 * k, 16)]
                    bv = br[e, pl.ds(16 * k, 16)]
                    ob[e, pl.ds(16 * k, 16)] = av + bv
            return ce
        lax.fori_loop(0, SC_CH // 16, _egroup, 0)
        pltpu.async_copy(ob, out_h.at[pl.ds(base, SC_CH)], sem_o).wait()
        return cr
    lax.fori_loop(0, SC_NCH_G, _chunk, 0)


def _sc_scatter_body(h_h, dst_h, out_h, di0, di1, mi, hb0, hb1, zb, acc,
                     sem_d0, sem_d1, sem_h0, sem_h1):
    core = lax.axis_index("c")
    sid = lax.axis_index("s")
    io16 = lax.iota(jnp.int32, 16)
    zero16 = jnp.zeros((16,), F32)
    tbase = sid * (SC_NCH_S * SC_CHS)
    dis = (di0, di1)
    hbs = (hb0, hb1)
    sds = (sem_d0, sem_d1)
    shs = (sem_h0, sem_h1)

    def _start(c, b):
        base = tbase + c * SC_CHS
        pltpu.async_copy(dst_h.at[pl.ds(base, SC_CHS)], dis[b], sds[b])
        pltpu.async_copy(h_h.at[pl.ds(base, SC_CHS)], hbs[b], shs[b])

    def _wait(b):
        pltpu.make_async_copy(dst_h.at[pl.ds(0, SC_CHS)], dis[b],
                              sds[b]).wait()
        pltpu.make_async_copy(h_h.at[pl.ds(0, SC_CHS)], hbs[b],
                              shs[b]).wait()

    def _zrow(r, cr):
        for t in range(AUG // 16):
            zb[r, pl.ds(16 * t, 16)] = zero16
        return cr
    lax.fori_loop(0, SC_CHS, _zrow, 0)

    for ps in range(2):
        q = core * 2 + ps
        lo = q * QROWS
        zbase = sid * (ACC_ROWS // 16)
        for z in range(12):
            pltpu.sync_copy(zb, acc.at[pl.ds(zbase + z * SC_CHS, SC_CHS)])
        pltpu.sync_copy(zb.at[pl.ds(0, 20)],
                        acc.at[pl.ds(zbase + 12 * SC_CHS, 20)])
        plsc.subcore_barrier()

        def _consume(c, b):
            base = tbase + c * SC_CHS

            def _gmap(g, cg):
                d = dis[b][pl.ds(g * 16, 16)]
                local = d - lo
                eg = io16 + (base + g * 16)
                ok = (local >= 0) & (local < QROWS) & (eg < E_RC_TOTAL)
                trash = QROWS + (io16 & 7)
                mi[pl.ds(g * 16, 16)] = jnp.where(ok, local, trash)
                return cg
            lax.fori_loop(0, SC_CHS // 16, _gmap, 0)
            _wait(b)
            pltpu.sync_copy(hbs[b], acc.at[mi], add=True)

        def _pair(i, cr):
            c0 = i * 2
            _start(c0 + 1, 1)
            _consume(c0, 0)
            _start(jnp.minimum(c0 + 2, SC_NCH_S - 1), 0)
            _consume(c0 + 1, 1)
            return cr

        _start(0, 0)
        lax.fori_loop(0, SC_NCH_S // 2, _pair, 0)
        _wait(0)
        plsc.subcore_barrier()

        @pl.when(sid == 0)
        def _flush():
            pltpu.sync_copy(acc.at[pl.ds(0, QROWS)], out_h.at[q])
        plsc.subcore_barrier()


E_RC_TOTAL = 800000
SC_NCH_G = E_PAD // (32 * SC_CH)
SC_NCH_S = E_PAD // (16 * SC_CHS)
QROWS = 12544
NPASS = 4
ACC_ROWS = 12608


def _ln_aug_kernel(h_ref, e0_ref, e1_ref, w0_ref, w1_ref, g_ref, b_ref,
                   o_ref):
    h = (h_ref[:, :EMB] + e0_ref[...] * w0_ref[...]
         + e1_ref[...] * w1_ref[...])
    ln = _relu(_ln_rows(h, g_ref[...], b_ref[...]))
    C = ln.shape[0]
    o_ref[...] = jnp.concatenate(
        [ln, jnp.ones((C, 1), F32), jnp.zeros((C, AUG - EMB - 1), F32)],
        axis=-1)


def _sc_edge_stage(A, B, srcp, dstp, ev0p, ev1p, w0p, w1p, ln_g, ln_b):
    mesh = plsc.VectorSubcoreMesh(core_axis_name="c", subcore_axis_name="s")

    gather = functools.partial(
        pl.kernel, mesh=mesh,
        out_type=jax.ShapeDtypeStruct((E_PAD, AUG), F32),
        scratch_types=[
            pltpu.VMEM((SC_CH,), jnp.int32),
            pltpu.VMEM((SC_CH,), jnp.int32),
            pltpu.VMEM((SC_CH, AUG), F32),
            pltpu.VMEM((SC_CH, AUG), F32),
            pltpu.VMEM((SC_CH, AUG), F32),
            pltpu.SemaphoreType.DMA,
            pltpu.SemaphoreType.DMA,
            pltpu.SemaphoreType.DMA,
        ])(_sc_gather_body)
    h = gather(A, B, srcp, dstp)

    CH2 = 2048
    haug = pl.pallas_call(
        _ln_aug_kernel,
        grid=(E_PAD // CH2,),
        in_specs=[pl.BlockSpec((CH2, AUG), lambda i: (i, 0)),
                  pl.BlockSpec((CH2, 1), lambda i: (i, 0)),
                  pl.BlockSpec((CH2, 1), lambda i: (i, 0)),
                  pl.BlockSpec((1, EMB), lambda i: (0, 0)),
                  pl.BlockSpec((1, EMB), lambda i: (0, 0)),
                  pl.BlockSpec((1, EMB), lambda i: (0, 0)),
                  pl.BlockSpec((1, EMB), lambda i: (0, 0))],
        out_specs=pl.BlockSpec((CH2, AUG), lambda i: (i, 0)),
        out_shape=jax.ShapeDtypeStruct((E_PAD, AUG), F32),
    )(h, ev0p[:, None], ev1p[:, None], w0p[None, :], w1p[None, :],
      ln_g[None, :], ln_b[None, :])

    scatter = functools.partial(
        pl.kernel, mesh=mesh,
        out_type=jax.ShapeDtypeStruct((NPASS, QROWS, AUG), F32),
        scratch_types=[
            pltpu.VMEM((SC_CHS,), jnp.int32),
            pltpu.VMEM((SC_CHS,), jnp.int32),
            pltpu.VMEM((SC_CHS,), jnp.int32),
            pltpu.VMEM((SC_CHS, AUG), F32),
            pltpu.VMEM((SC_CHS, AUG), F32),
            pltpu.VMEM((SC_CHS, AUG), F32),
            pltpu.VMEM_SHARED((ACC_ROWS, AUG), F32),
            pltpu.SemaphoreType.DMA,
            pltpu.SemaphoreType.DMA,
            pltpu.SemaphoreType.DMA,
            pltpu.SemaphoreType.DMA,
        ])(_sc_scatter_body)
    out = scatter(haug, dstp)
    return out.reshape(NPASS * QROWS, AUG)[:50000]


def kernel(x_rows, x_cols, x_sepas, edge_index_rowcols, edge_vals_rowcols,
           edge_index_sepa_cols, edge_vals_sepa_cols, edge_index_sepa_rows,
           edge_vals_sepa_rows, edge_index_sepa_self, edge_vals_sepa_self,
           params):
    p = params
    ei_rc = edge_index_rowcols.astype(jnp.int32)
    ei_sc = edge_index_sepa_cols.astype(jnp.int32)
    ei_sr = edge_index_sepa_rows.astype(jnp.int32)
    ei_ss = edge_index_sepa_self.astype(jnp.int32)

    E_RC = ei_rc.shape[1]
    E_SC = ei_sc.shape[1]
    E_SR = ei_sr.shape[1]

    evT_rc = edge_vals_rowcols.T.reshape(2, E_RC // 128, 128)
    Wp_rc, c_rc = _edge_fold(evT_rc, E_RC, p['en_rowcols_g'],
                             p['en_rowcols_b'], p['c2r']['e_W'])
    Wp_rc2, c_rc2 = _edge_fold(evT_rc, E_RC, p['en_rowcols_g'],
                               p['en_rowcols_b'], p['r2c']['e_W'])

    def _pad128(v):
        E = v.shape[0]
        R = -(-E // 128) * 128
        return jnp.pad(v, (0, R - E)).reshape(1, R // 128, 128)

    Wp_sc, c_sc = _edge_fold(_pad128(edge_vals_sepa_cols[:, 0]), E_SC,
                             p['en_sepas_g'], p['en_sepas_b'], p['c2s']['e_W'])
    Wp_sr, c_sr = _edge_fold(_pad128(edge_vals_sepa_rows[:, 0]), E_SR,
                             p['en_rows_g'], p['en_rows_b'], p['s2r']['e_W'])
    Wp_r2s, c_r2s = _edge_fold(_pad128(edge_vals_sepa_rows[:, 0]), E_SR,
                               p['en_rows_g'], p['en_rows_b'], p['r2s']['e_W'])

    row0, A_c2r, _ = _prologue(
        x_rows, p['row_bn_g'], p['row_bn_b'], p['row_W1'], p['row_b1'],
        p['row_W2'], p['row_b2'],
        p['c2r']['l_W'], p['c2r']['l_b'] + c_rc,
        jnp.zeros((EMB, EMB), F32), jnp.zeros((EMB,), F32))
    col0, B_c2r, A_r2c = _prologue(
        x_cols, p['col_bn_g'], p['col_bn_b'], p['col_W1'], p['col_b1'],
        p['col_W2'], p['col_b2'],
        p['c2r']['r_W'], jnp.zeros((EMB,), F32),
        p['r2c']['l_W'], p['r2c']['l_b'] + c_rc2)

    padn = E_PAD - E_RC
    srcp = jnp.pad(ei_rc[1], (0, padn))
    dstp = jnp.pad(ei_rc[0], (0, padn))
    ev0p = jnp.pad(edge_vals_rowcols[:, 0], (0, padn))
    ev1p = jnp.pad(edge_vals_rowcols[:, 1], (0, padn))
    hs_c2r = _sc_edge_stage(A_c2r, B_c2r, srcp, dstp, ev0p, ev1p,
                            Wp_rc[0], Wp_rc[1],
                            p['c2r']['fin_g'], p['c2r']['fin_bln'])
    row1, B_r2c, _ = _post(hs_c2r, row0, p['c2r'],
                           p['r2c']['r_W'], jnp.zeros((EMB,), F32))

    hs_r2c = _sc_edge_stage(A_r2c, B_r2c, dstp, srcp, ev0p, ev1p,
                            Wp_rc2[0], Wp_rc2[1],
                            p['r2c']['fin_g'], p['r2c']['fin_bln'])
    col1, _, colsum = _post(hs_r2c, col0, p['r2c'],
                            jnp.zeros((EMB, EMB), F32), jnp.zeros((EMB,), F32))

    sep_b = p['sepa_ln_b'][None, :]
    sep0_row = _relu(_relu(sep_b @ p['sepa_W1'] + p['sepa_b1'])
                     @ p['sepa_W2'] + p['sepa_b2'])
    sep0 = jnp.broadcast_to(sep0_row, (SROWS, EMB)) * (
        (jnp.arange(SROWS) < 17).astype(F32)[:, None])
    A32_c2s = _proj32(sep0, p['c2s']['l_W'], p['c2s']['l_b'] + c_sc)
    B32_c2s = _proj32(jnp.pad(col1[:17], ((0, SROWS - 17), (0, 0))),
                      p['c2s']['r_W'], jnp.zeros((EMB,), F32))

    src_sc = ei_sc[1][:, None]
    dst_sc = ei_sc[0][:, None]
    hs_c2s = _small_edge(A32_c2s, B32_c2s, src_sc, dst_sc,
                         edge_vals_sepa_cols, Wp_sc[0][None, :],
                         p['c2s']['fin_g'], p['c2s']['fin_bln'])
    sep1, _, B32_s2r = _small_post(
        hs_c2s, sep0, p['c2s'],
        jnp.zeros((EMB, EMB), F32), jnp.zeros((EMB,), F32), p['s2r']['r_W'])
    row1_17 = jnp.pad(row1[:17], ((0, SROWS - 17), (0, 0)))
    A32_s2r = _proj32(row1_17, p['s2r']['l_W'], p['s2r']['l_b'] + c_sr)

    src_sr = ei_sr[0][:, None]
    dst_sr = ei_sr[1][:, None]
    hs_s2r17 = _small_edge(A32_s2r, B32_s2r, src_sr, dst_sr,
                           edge_vals_sepa_rows, Wp_sr[0][None, :],
                           p['s2r']['fin_g'], p['s2r']['fin_bln'])
    hs_s2r = jnp.concatenate(
        [hs_s2r17[:17], jnp.zeros((row1.shape[0] - 17, AUG), F32)], axis=0)
    row2, _, _ = _post(hs_s2r, row1, p['s2r'],
                       jnp.zeros((EMB, EMB), F32), jnp.zeros((EMB,), F32))

    row2_17 = jnp.pad(row2[:17], ((0, SROWS - 17), (0, 0)))
    A32_r2s = _proj32(sep1, p['r2s']['l_W'], p['r2s']['l_b'] + c_r2s)
    B32_r2s = _proj32(row2_17, p['r2s']['r_W'], jnp.zeros((EMB,), F32))

    hs_r2s = _small_edge(A32_r2s, B32_r2s, dst_sr, src_sr,
                         edge_vals_sepa_rows, Wp_r2s[0][None, :],
                         p['r2s']['fin_g'], p['r2s']['fin_bln'])
    sep2, _, _ = _small_post(hs_r2s, sep1, p['r2s'],
                             jnp.zeros((EMB, EMB), F32),
                             jnp.zeros((EMB,), F32),
                             jnp.zeros((EMB, EMB), F32))

    E_SS = ei_ss.shape[1]
    PSS = -(-E_SS // 8) * 8
    src_ss = jnp.pad(ei_ss[0], (0, PSS - E_SS),
                     constant_values=SROWS - 1)[:, None]
    dst_ss = jnp.pad(ei_ss[1], (0, PSS - E_SS),
                     constant_values=SROWS - 1)[:, None]
    ev_ss = jnp.pad(edge_vals_sepa_self[:, 0], (0, PSS - E_SS))[:, None]
    xs32 = jnp.pad(x_sepas, ((0, SROWS - 17), (0, 0)))
    sattmean = _tconv_satt_mean(sep2, xs32, src_ss, dst_ss, ev_ss, p)

    N = row2.shape[0]
    rattsum = _relu_matsum(row2, p['ro_W'], p['ro_b'])

    return _final(sattmean, rattsum / N, colsum / N, p)
